# Initial kernel scaffold; baseline (speedup 1.0000x reference)
import jax
import jax.numpy as jnp
from jax.experimental import pallas as pl


def kernel(sequence, token_table):
    raise NotImplementedError("write your pallas kernel here")


# speedup vs baseline: 1.3762x; 1.3762x over previous
import functools

import numpy as np
import jax
import jax.numpy as jnp
from jax import lax
from jax.experimental import pallas as pl
from jax.experimental.pallas import tpu as pltpu
from jax.experimental.pallas import tpu_sc as plsc

_SEQ = 200
_D = 32
_BATCH = 4096
_NW = 32
_ROWS_PER_W = _BATCH * _SEQ // _NW
_CHUNK = 800
_NCHUNK = _ROWS_PER_W // _CHUNK
_GSZ = 100
_NG = _CHUNK // _GSZ
_NSTREAM = _ROWS_PER_W // _GSZ


def _positional_embedding():
    pos = np.arange(_SEQ, dtype=np.float32)[:, None]
    exp_sin = np.arange(0, _D, 2, dtype=np.float32) / _D * 2.0
    exp_cos = np.arange(1, _D + 1, 2, dtype=np.float32) / _D * 2.0
    sins = np.sin(pos / np.power(10000.0, exp_sin))
    coss = np.cos(pos / np.power(10000.0, exp_cos))
    pe = np.stack([sins, coss], axis=2).reshape(_SEQ, _D)
    return jnp.asarray(pe, dtype=jnp.float32)


def _body(idx_hbm, pe_hbm, table_hbm, out_hbm, idx_v, pe_v, rows_v, sem):
    wid = lax.axis_index("s") * 2 + lax.axis_index("c")
    pltpu.sync_copy(idx_hbm.at[wid], idx_v)
    for q in range(_CHUNK // _SEQ):
        pltpu.sync_copy(pe_hbm, pe_v.at[pl.ds(q * _SEQ, _SEQ)])
    out_base = wid * _ROWS_PER_W

    def chunk_body(ci, carry):
        for j in range(_NG):
            pltpu.make_async_copy(
                table_hbm.at[idx_v.at[ci * _NG + j]],
                rows_v.at[pl.ds(j * _GSZ, _GSZ)],
                sem,
            ).start()
        for j in range(_NG):
            pltpu.make_async_copy(
                table_hbm.at[idx_v.at[ci * _NG + j]],
                rows_v.at[pl.ds(j * _GSZ, _GSZ)],
                sem,
            ).wait()

        def add_body(i, c2):
            base = i * 8
            for k in range(8):
                r = base + k
                plsc.addupdate(rows_v.at[r, pl.ds(0, 16)], pe_v[r, pl.ds(0, 16)])
                plsc.addupdate(rows_v.at[r, pl.ds(16, 16)], pe_v[r, pl.ds(16, 16)])
            return c2

        lax.fori_loop(0, _CHUNK // 8, add_body, 0, unroll=False)

        pltpu.sync_copy(rows_v, out_hbm.at[pl.ds(out_base + ci * _CHUNK, _CHUNK)])
        return carry

    lax.fori_loop(0, _NCHUNK, chunk_body, 0, unroll=False)


@jax.jit
def _embed(idx_grouped, pe, token_table):
    mesh = plsc.VectorSubcoreMesh(core_axis_name="c", subcore_axis_name="s")
    run = functools.partial(
        pl.kernel,
        mesh=mesh,
        out_type=jax.ShapeDtypeStruct((_BATCH * _SEQ, _D), jnp.float32),
        scratch_types=[
            pltpu.VMEM((_NSTREAM, _GSZ), jnp.int32),
            pltpu.VMEM((_CHUNK, _D), jnp.float32),
            pltpu.VMEM((_CHUNK, _D), jnp.float32),
            pltpu.SemaphoreType.DMA,
        ],
        compiler_params=pltpu.CompilerParams(use_tc_tiling_on_sc=False),
    )(_body)
    return run(idx_grouped, pe, token_table)


def kernel(sequence, token_table):
    idx_grouped = sequence.reshape(_NW, _NSTREAM, _GSZ)
    pe = _positional_embedding()
    flat = _embed(idx_grouped, pe, token_table)
    return flat.reshape(_BATCH, _SEQ, _D)
